# baseline (speedup 1.0000x reference)
import functools

import jax
import jax.numpy as jnp
from jax import lax
from jax.experimental import pallas as pl
from jax.experimental.pallas import tpu as pltpu
from jax.experimental.pallas import tpu_sc as plsc

VOCAB = 1000000
EMB_DIM = 32
BATCH = 16384

_info = plsc.get_sparse_core_info()
_NS = _info.num_subcores
_NW = _NS
_B_PER_W = BATCH // _NW
_CH = 512
_GRP = 16
_ROWS = VOCAB // 4


def _gather_kernel(tr_hbm, idx_hbm, out_hbm, idx_v, idx4_v, rows_v, sel_v, sem):
    wid = lax.axis_index("s")
    base = wid * _B_PER_W

    for h in range(_B_PER_W // _CH):
        pltpu.sync_copy(idx_hbm.at[pl.ds(base + h * _CH, _CH)], idx_v)

        def prep(g, carry):
            vec = jnp.clip(idx_v[pl.ds(g * _GRP, _GRP)], 0, VOCAB - 1)
            idx4_v[pl.ds(g * _GRP, _GRP)] = lax.shift_right_logical(vec, 2)
            return carry

        lax.fori_loop(0, _CH // _GRP, prep, 0)

        pltpu.async_copy(tr_hbm.at[idx4_v], rows_v, sem).wait()

        def select(g, carry):
            vec = jnp.clip(idx_v[pl.ds(g * _GRP, _GRP)], 0, VOCAB - 1)
            lane0_vec = jnp.bitwise_and(vec, 3) * 32
            for j in range(_GRP):
                lane0 = lax.index_in_dim(lane0_vec, j, axis=0, keepdims=False)
                t = g * _GRP + j
                lo = rows_v[t, pl.ds(lane0, 16)]
                hi = rows_v[t, pl.ds(lane0 + 16, 16)]
                r = g * 4 + (j >> 2)
                c = (j & 3) * 32
                sel_v[r, pl.ds(c, 16)] = lo
                sel_v[r, pl.ds(c + 16, 16)] = hi
            return carry

        lax.fori_loop(0, _CH // _GRP, select, 0)
        pltpu.sync_copy(
            sel_v, out_hbm.at[pl.ds(wid * (_B_PER_W * EMB_DIM // 128) + h * 128, 128)]
        )


@jax.jit
def kernel(customer_id, table):
    idx = customer_id.astype(jnp.int32)
    tr = table.reshape(_ROWS, 128)
    mesh = plsc.VectorSubcoreMesh(
        core_axis_name="c", subcore_axis_name="s", num_cores=1
    )
    f = functools.partial(
        pl.kernel,
        mesh=mesh,
        out_type=jax.ShapeDtypeStruct((BATCH * EMB_DIM // 128, 128), jnp.float32),
        scratch_types=[
            pltpu.VMEM((_CH,), jnp.int32),
            pltpu.VMEM((_CH,), jnp.int32),
            pltpu.VMEM((_CH, 128), jnp.float32),
            pltpu.VMEM((128, 128), jnp.float32),
            pltpu.SemaphoreType.DMA,
        ],
    )(_gather_kernel)
    return f(tr, idx).reshape(BATCH, EMB_DIM)


# speedup vs baseline: 1.0195x; 1.0195x over previous
import functools

import jax
import jax.numpy as jnp
from jax import lax
from jax.experimental import pallas as pl
from jax.experimental.pallas import tpu as pltpu
from jax.experimental.pallas import tpu_sc as plsc

VOCAB = 1000000
EMB_DIM = 32
BATCH = 16384

_info = plsc.get_sparse_core_info()
_NC, _NS = _info.num_cores, _info.num_subcores
_NW = _NC * _NS
_B_PER_W = BATCH // _NW


def _gather_kernel(table_hbm, idx_hbm, out_hbm, idx_v, rows_v, sem):
    wid = lax.axis_index("s") * _NC + lax.axis_index("c")
    base = wid * _B_PER_W
    pltpu.sync_copy(idx_hbm.at[pl.ds(base, _B_PER_W)], idx_v)
    pltpu.async_copy(table_hbm.at[idx_v], rows_v, sem).wait()
    pltpu.sync_copy(rows_v, out_hbm.at[pl.ds(base, _B_PER_W)])


@jax.jit
def kernel(customer_id, table):
    idx = customer_id.astype(jnp.int32)
    mesh = plsc.VectorSubcoreMesh(core_axis_name="c", subcore_axis_name="s")
    f = functools.partial(
        pl.kernel,
        mesh=mesh,
        out_type=jax.ShapeDtypeStruct((BATCH, EMB_DIM), jnp.float32),
        scratch_types=[
            pltpu.VMEM((_B_PER_W,), jnp.int32),
            pltpu.VMEM((_B_PER_W, EMB_DIM), jnp.float32),
            pltpu.SemaphoreType.DMA,
        ],
        compiler_params=pltpu.CompilerParams(use_tc_tiling_on_sc=False),
    )(_gather_kernel)
    return f(table, idx)
